# baseline (speedup 1.0000x reference)
import jax
import jax.numpy as jnp
from jax import lax
from jax.experimental import pallas as pl
from jax.experimental.pallas import tpu as pltpu
from jax.experimental.pallas import tpu_sc as plsc

N_VOCAB = 100000
N_EMBD = 128
N_TOKENS = 200
BATCH = 1024

_NC = 2
_NS = 16
_NW = _NC * _NS
_BPW = BATCH // _NW
_HL = 104
_PT = 2 * _HL


def _body(x_ref, tab_ref, pos_ref, out_ref,
          idx_v, pos_sh,
          b0, b1, b2, b3,
          pf0, pf1, pf2, pf3, sg0, sg1, sg2, sg3, so0, so1, so2, so3):
    sid = lax.axis_index("s")
    wid = sid * _NC + lax.axis_index("c")
    base = wid * _BPW
    bufs = [b0, b1, b2, b3]
    spf = [pf0, pf1, pf2, pf3]
    sgs = [sg0, sg1, sg2, sg3]
    sos = [so0, so1, so2, so3]

    pltpu.sync_copy(x_ref.at[pl.ds(wid * _BPW, _BPW)], idx_v)

    @pl.when(sid == 0)
    def _():
        pltpu.sync_copy(pos_ref, pos_sh)

    plsc.subcore_barrier()

    def pf(r, q):
        pltpu.async_copy(pos_sh, bufs[q], spf[q])

    def wpf_g(r, q):
        pltpu.make_async_copy(pos_sh, bufs[q], spf[q]).wait()
        pltpu.async_copy(tab_ref.at[idx_v.at[r - base, 0]],
                         bufs[q].at[pl.ds(0, _HL)], sgs[q], add=True)
        pltpu.async_copy(tab_ref.at[idx_v.at[r - base, 1]],
                         bufs[q].at[pl.ds(_HL, _HL)], sgs[q], add=True)

    def wg_o(r, q):
        pltpu.make_async_copy(tab_ref.at[idx_v.at[r - base, 0]],
                              bufs[q].at[pl.ds(0, _HL)], sgs[q]).wait()
        pltpu.make_async_copy(tab_ref.at[idx_v.at[r - base, 1]],
                              bufs[q].at[pl.ds(_HL, _HL)], sgs[q]).wait()
        pltpu.async_copy(bufs[q].at[pl.ds(0, N_TOKENS)], out_ref.at[r], sos[q])

    def wo(r, q):
        pltpu.make_async_copy(bufs[q].at[pl.ds(0, N_TOKENS)],
                              out_ref.at[r], sos[q]).wait()

    pf(base + 0, 0)
    pf(base + 1, 1)
    wpf_g(base + 0, 0)
    pf(base + 2, 2)
    wpf_g(base + 1, 1)
    wg_o(base + 0, 0)
    pf(base + 3, 3)
    wpf_g(base + 2, 2)
    wg_o(base + 1, 1)

    def group(g, carry):
        r0 = base + 4 * g
        for q in range(4):
            r = r0 + q
            wo(r - 4, q)
            pf(r, q)
            wpf_g(r - 1, (q + 3) % 4)
            wg_o(r - 2, (q + 2) % 4)
        return carry

    lax.fori_loop(1, _BPW // 4, group, 0)

    last = base + _BPW - 1
    wpf_g(last, 3)
    wg_o(last - 1, 2)
    wg_o(last, 3)
    wo(last - 3, 0)
    wo(last - 2, 1)
    wo(last - 1, 2)
    wo(last, 3)


@jax.jit
def kernel(x, embedding_table, positional_embedding):
    xi = x.astype(jnp.int32)
    x4 = jnp.stack(
        [xi[:, :_HL], jnp.pad(xi[:, _HL:], ((0, 0), (0, _PT - N_TOKENS)))],
        axis=1)
    pos_pad = jnp.pad(positional_embedding,
                      ((0, _PT - N_TOKENS), (0, 0)))
    mesh = plsc.VectorSubcoreMesh(
        core_axis_name="c", subcore_axis_name="s",
        num_cores=_NC, num_subcores=_NS)
    return pl.kernel(
        _body,
        out_type=jax.ShapeDtypeStruct((BATCH, N_TOKENS, N_EMBD), jnp.float32),
        mesh=mesh,
        scratch_types=[
            pltpu.VMEM((_BPW, 2, _HL), jnp.int32),
            pltpu.VMEM_SHARED((_PT, N_EMBD), jnp.float32),
        ] + [pltpu.VMEM((_PT, N_EMBD), jnp.float32)] * 4
          + [pltpu.SemaphoreType.DMA] * 12,
    )(x4, embedding_table, pos_pad)


# speedup vs baseline: 2.2552x; 2.2552x over previous
import jax
import jax.numpy as jnp
from jax import lax
from jax.experimental import pallas as pl
from jax.experimental.pallas import tpu as pltpu
from jax.experimental.pallas import tpu_sc as plsc

N_VOCAB = 100000
N_EMBD = 128
N_TOKENS = 200
BATCH = 1024

_NC = 2
_NS = 16
_NW = _NC * _NS
_BPW = BATCH // _NW
_H = N_TOKENS // 2
_NH = 2 * _BPW


def _body(x_ref, tab_ref, p0_ref, p1_ref, out_ref,
          idx_v, p0_sh, p1_sh,
          b0, b1, b2, b3,
          pf0, pf1, pf2, pf3, sg0, sg1, sg2, sg3, so0, so1, so2, so3):
    sid = lax.axis_index("s")
    wid = sid * _NC + lax.axis_index("c")
    base = wid * _BPW
    bufs = [b0, b1, b2, b3]
    spf = [pf0, pf1, pf2, pf3]
    sgs = [sg0, sg1, sg2, sg3]
    sos = [so0, so1, so2, so3]
    pos_sh = [p0_sh, p1_sh]

    pltpu.sync_copy(x_ref.at[pl.ds(wid * _BPW, _BPW)], idx_v)

    @pl.when(sid == 0)
    def _():
        pltpu.sync_copy(p0_ref, p0_sh)
        pltpu.sync_copy(p1_ref, p1_sh)

    plsc.subcore_barrier()

    def pf(j, q):
        pltpu.async_copy(pos_sh[q % 2], bufs[q], spf[q])

    def wpf_g(j, q):
        pltpu.make_async_copy(pos_sh[q % 2], bufs[q], spf[q]).wait()
        pltpu.async_copy(tab_ref.at[idx_v.at[j // 2, q % 2]], bufs[q],
                         sgs[q], add=True)

    def wg_o(j, q):
        pltpu.make_async_copy(tab_ref.at[idx_v.at[j // 2, q % 2]], bufs[q],
                              sgs[q]).wait()
        pltpu.async_copy(bufs[q], out_ref.at[base + j // 2, q % 2], sos[q])

    def wo(j, q):
        pltpu.make_async_copy(bufs[q], out_ref.at[base + j // 2, q % 2],
                              sos[q]).wait()

    pf(0, 0)
    pf(1, 1)
    wpf_g(0, 0)
    pf(2, 2)
    wpf_g(1, 1)
    wg_o(0, 0)
    pf(3, 3)
    wpf_g(2, 2)
    wg_o(1, 1)

    def group(g, carry):
        j0 = 4 * g
        for q in range(4):
            j = j0 + q
            wo(j - 4, q)
            pf(j, q)
            wpf_g(j - 1, (q + 3) % 4)
            wg_o(j - 2, (q + 2) % 4)
        return carry

    lax.fori_loop(1, _NH // 4, group, 0)

    last = _NH - 1
    wpf_g(last, 3)
    wg_o(last - 1, 2)
    wg_o(last, 3)
    wo(last - 3, 0)
    wo(last - 2, 1)
    wo(last - 1, 2)
    wo(last, 3)


@jax.jit
def kernel(x, embedding_table, positional_embedding):
    x3 = x.reshape(BATCH, 2, _H).astype(jnp.int32)
    p0 = positional_embedding[:_H]
    p1 = positional_embedding[_H:]
    mesh = plsc.VectorSubcoreMesh(
        core_axis_name="c", subcore_axis_name="s",
        num_cores=_NC, num_subcores=_NS)
    out = pl.kernel(
        _body,
        out_type=jax.ShapeDtypeStruct((BATCH, 2, _H, N_EMBD), jnp.float32),
        mesh=mesh,
        scratch_types=[
            pltpu.VMEM((_BPW, 2, _H), jnp.int32),
            pltpu.VMEM_SHARED((_H, N_EMBD), jnp.float32),
            pltpu.VMEM_SHARED((_H, N_EMBD), jnp.float32),
        ] + [pltpu.VMEM((_H, N_EMBD), jnp.float32)] * 4
          + [pltpu.SemaphoreType.DMA] * 12,
    )(x3, embedding_table, p0, p1)
    return out.reshape(BATCH, N_TOKENS, N_EMBD)
